# Initial kernel scaffold; baseline (speedup 1.0000x reference)
import jax
import jax.numpy as jnp
from jax.experimental import pallas as pl


def kernel(node_feats, edge_attr, triplet_feats, edge_index, edge_index_lg, eu_Wsrc_w, eu_Wsrc_b, eu_Wdst_w, eu_Wdst_b, eu_Wai_w, eu_Wai_b, eu_Wbj_w, eu_Wbj_b, eu_Wcij_w, eu_Wcij_b, eu_bn_n_g, eu_bn_n_b, eu_bn_e_g, eu_bn_e_b, nu_Wsrc_w, nu_Wsrc_b, nu_Wdst_w, nu_Wdst_b, nu_Wai_w, nu_Wai_b, nu_Wbj_w, nu_Wbj_b, nu_Wcij_w, nu_Wcij_b, nu_bn_n_g, nu_bn_n_b, nu_bn_e_g, nu_bn_e_b):
    raise NotImplementedError("write your pallas kernel here")


# speedup vs baseline: 1.0002x; 1.0002x over previous
import jax
import jax.numpy as jnp
from jax.experimental import pallas as pl

PN = ["Wsrc_w", "Wsrc_b", "Wdst_w", "Wdst_b", "Wai_w", "Wai_b", "Wbj_w", "Wbj_b", "Wcij_w", "Wcij_b", "bn_n_g", "bn_n_b", "bn_e_g", "bn_e_b"]


def _lin(x, w, b):
    return x @ w.T + b


def _bn(x, g, b):
    mu = jnp.mean(x, axis=0)
    var = jnp.var(x, axis=0)
    return g * (x - mu) / jnp.sqrt(var + 1e-5) + b


def _silu(x):
    return x * jax.nn.sigmoid(x)


def _egg(x, ea, ei, p):
    (Wsw, Wsb, Wdw, Wdb, Waw, Wab, Wbw, Wbb, Wcw, Wcb, gn, bn, ge, be) = p
    n = x.shape[0]
    i = ei[0]
    j = ei[1]
    sigma = jax.nn.sigmoid(ea)
    ssum = jax.ops.segment_sum(sigma, i, num_segments=n)
    ehat = sigma / (ssum[i] + 1e-6)
    dest = jax.ops.segment_sum(ehat * _lin(x[i], Wdw, Wdb), j, num_segments=n)
    new_x = x + _silu(_bn(_lin(x, Wsw, Wsb) + dest, gn, bn))
    new_ea = ea + _silu(_bn(_lin(x[i], Waw, Wab) + _lin(x[j], Wbw, Wbb) + _lin(ea, Wcw, Wcb), ge, be))
    return new_x, new_ea


def kernel(node_feats, edge_attr, triplet_feats, edge_index, edge_index_lg, eu_Wsrc_w, eu_Wsrc_b, eu_Wdst_w, eu_Wdst_b, eu_Wai_w, eu_Wai_b, eu_Wbj_w, eu_Wbj_b, eu_Wcij_w, eu_Wcij_b, eu_bn_n_g, eu_bn_n_b, eu_bn_e_g, eu_bn_e_b, nu_Wsrc_w, nu_Wsrc_b, nu_Wdst_w, nu_Wdst_b, nu_Wai_w, nu_Wai_b, nu_Wbj_w, nu_Wbj_b, nu_Wcij_w, nu_Wcij_b, nu_bn_n_g, nu_bn_n_b, nu_bn_e_g, nu_bn_e_b):
    kw = dict(locals())
    pe = tuple(kw["eu_" + n] for n in PN)
    pn = tuple(kw["nu_" + n] for n in PN)
    message, trip = _egg(edge_attr, triplet_feats, edge_index_lg, pe)
    nf, ea = _egg(node_feats, message, edge_index, pn)
    return (nf, ea, trip)


# speedup vs baseline: 1.3731x; 1.3727x over previous
import jax
import jax.numpy as jnp
from jax import lax
from jax.experimental import pallas as pl
from jax.experimental.pallas import tpu as pltpu
from jax.experimental.pallas import tpu_sc as plsc

D = 128
L = 16
NT = 16
NC = 2
B = 112
SB = 2000

_mesh = lambda: plsc.VectorSubcoreMesh(core_axis_name="c", subcore_axis_name="s")
_params = lambda: pltpu.CompilerParams(needs_layout_passes=False)


def _zero_rows(buf, nrows):
    def body(r, _):
        for k in range(D // L):
            buf[r, pl.ds(k * L, L)] = jnp.zeros((L,), jnp.float32)
        return 0
    lax.fori_loop(0, nrows, body, 0)


def _init_i32(buf, nwords):
    assert nwords % L == 0
    def body(v, _):
        buf[pl.ds(v * L, L)] = jnp.zeros((L,), jnp.int32)
        return 0
    lax.fori_loop(0, nwords // L, body, 0)


def _copy_span(src, dst, span, chunk_rows, advance_src=True):
    o = 0
    while o < span:
        sz = min(chunk_rows, span - o)
        so = o if advance_src else 0
        pltpu.sync_copy(src.at[pl.ds(so, sz)], dst.at[pl.ds(o, sz)])
        o += sz


def _scatter_batch(rows, dst_t, acc):
    for k in range(B // L):
        dvec = dst_t[pl.ds(k * L, L)]
        pltpu.sync_copy(rows.at[pl.ds(k * L, L)], acc.at[dvec], add=True)


def _set_dump_tail(dst_t, cnt, W):
    iota = lax.iota(jnp.int32, L)
    full_w = jnp.full((L,), W, jnp.int32)
    for k in range(B // L):
        plsc.store_scatter(dst_t, [cnt + k * L + iota], full_w)


def _shift_down(bufs):
    for buf in bufs:
        for k in range(B // L):
            buf[pl.ds(k * L, L)] = buf[pl.ds(B + k * L, L)]


def _scan_pass(idx_hbm, extra_hbm, ibuf, ebuf, pos_t, ext_t, dst_t, cnt_ref,
               s, chunk, wb, W, drain):
    iota = lax.iota(jnp.int32, L)
    cap = pos_t.shape[0]
    dump = jnp.int32(cap - 1)
    cnt_ref[0] = jnp.int32(0)

    for sb in range(chunk // SB):
        pltpu.sync_copy(idx_hbm.at[pl.ds(s * chunk + sb * SB, SB)], ibuf)
        if extra_hbm is not None:
            pltpu.sync_copy(extra_hbm.at[pl.ds(s * chunk + sb * SB, SB)], ebuf)
        gbase = s * chunk + sb * SB

        def blk(v, _):
            cnt = cnt_ref[0]
            i16 = ibuf[pl.ds(v * L, L)]
            m = (i16 >= wb) & (i16 < wb + W)
            mi = m.astype(jnp.int32)
            c1 = plsc.cumsum(mi)
            off = jnp.where(m, cnt + c1 - 1, dump)
            plsc.store_scatter(pos_t, [off], gbase + v * L + iota)
            plsc.store_scatter(dst_t, [off], i16 - wb)
            if extra_hbm is not None:
                plsc.store_scatter(ext_t, [off], ebuf[pl.ds(v * L, L)])
            cnt2 = cnt + jnp.sum(mi)
            cnt_ref[0] = cnt2

            @pl.when(cnt2 >= B)
            def _():
                drain()
                _shift_down([pos_t, dst_t] + ([ext_t] if extra_hbm is not None else []))
                cnt_ref[0] = cnt2 - B
            return 0

        lax.fori_loop(0, SB // L, blk, 0)

    cnt = cnt_ref[0]
    _set_dump_tail(dst_t, cnt, W)

    @pl.when(cnt > 0)
    def _():
        drain()


def _window_loop(P, W, acc, zrows, out_hbm, s, c, pass_body):
    T = (W // NT) & ~7
    Tl = W - (NT - 1) * T
    assert T > 0 and Tl > 0 and Tl % 8 == 0 and W % 8 == 0
    zr = zrows.shape[0]

    def one_pass(p, _):
        wb = (c * P + p) * W
        _zero_rows(zrows, zr)

        @pl.when(s < NT - 1)
        def _():
            _copy_span(zrows, acc.at[pl.ds(s * T, T)], T, zr, advance_src=False)

        @pl.when(s == NT - 1)
        def _():
            _copy_span(zrows, acc.at[pl.ds((NT - 1) * T, Tl)], Tl, zr, advance_src=False)
        plsc.subcore_barrier()
        pass_body(wb)
        plsc.subcore_barrier()

        @pl.when(s < NT - 1)
        def _():
            _copy_span(acc.at[pl.ds(s * T, T)], out_hbm.at[pl.ds(wb + s * T, T)], T, B)

        @pl.when(s == NT - 1)
        def _():
            _copy_span(acc.at[pl.ds((NT - 1) * T, Tl)],
                       out_hbm.at[pl.ds(wb + (NT - 1) * T, Tl)], Tl, B)
        plsc.subcore_barrier()
        return 0

    lax.fori_loop(0, P, one_pass, 0)


def _seg_sum_sc(Eu, n, W, P):
    chunk = Eu // NT
    cap = 2 * B + L

    def body(vals_hbm, idx_hbm, out_hbm, ibuf, pos_t, dst_t, rows, cnt_ref, acc):
        c = lax.axis_index("c")
        s = lax.axis_index("s")
        _init_i32(pos_t, cap)

        def pass_body(wb):
            def drain():
                pltpu.sync_copy(vals_hbm.at[pos_t.at[pl.ds(0, B)]], rows)
                _scatter_batch(rows, dst_t, acc)

            _scan_pass(idx_hbm, None, ibuf, None, pos_t, None, dst_t, cnt_ref,
                       s, chunk, wb, W, drain)

        _window_loop(P, W, acc, rows, out_hbm, s, c, pass_body)

    return pl.kernel(
        body,
        out_type=jax.ShapeDtypeStruct((n, D), jnp.float32),
        mesh=_mesh(),
        compiler_params=_params(),
        scratch_types=[
            pltpu.VMEM((SB,), jnp.int32),
            pltpu.VMEM((cap,), jnp.int32),
            pltpu.VMEM((cap,), jnp.int32),
            pltpu.VMEM((B, D), jnp.float32),
            pltpu.SMEM((8,), jnp.int32),
            pltpu.VMEM_SHARED((W + L, D), jnp.float32),
        ],
    )


def _dest_sc(Eu, n, W, P):
    chunk = Eu // NT
    cap = 2 * B + L

    def body(sig_hbm, i_hbm, j_hbm, ssum_hbm, xd_hbm, out_hbm,
             jbuf, ebuf, pos_t, iv_t, dst_t, srows, urows, cnt_ref, acc):
        c = lax.axis_index("c")
        s = lax.axis_index("s")
        _init_i32(pos_t, cap)
        _init_i32(iv_t, cap)

        def pass_body(wb):
            def drain():
                pltpu.sync_copy(sig_hbm.at[pos_t.at[pl.ds(0, B)]], srows)
                pltpu.sync_copy(ssum_hbm.at[iv_t.at[pl.ds(0, B)]], urows)

                def rowfn(r, _):
                    for k in range(D // L):
                        sl = pl.ds(k * L, L)
                        urows[r, sl] = srows[r, sl] / (urows[r, sl] + 1e-6)
                    return 0
                lax.fori_loop(0, B, rowfn, 0)
                pltpu.sync_copy(xd_hbm.at[iv_t.at[pl.ds(0, B)]], srows)

                def rowfn2(r, _):
                    for k in range(D // L):
                        sl = pl.ds(k * L, L)
                        urows[r, sl] = urows[r, sl] * srows[r, sl]
                    return 0
                lax.fori_loop(0, B, rowfn2, 0)
                _scatter_batch(urows, dst_t, acc)

            _scan_pass(j_hbm, i_hbm, jbuf, ebuf, pos_t, iv_t, dst_t, cnt_ref,
                       s, chunk, wb, W, drain)

        _window_loop(P, W, acc, urows, out_hbm, s, c, pass_body)

    return pl.kernel(
        body,
        out_type=jax.ShapeDtypeStruct((n, D), jnp.float32),
        mesh=_mesh(),
        compiler_params=_params(),
        scratch_types=[
            pltpu.VMEM((SB,), jnp.int32),
            pltpu.VMEM((SB,), jnp.int32),
            pltpu.VMEM((cap,), jnp.int32),
            pltpu.VMEM((cap,), jnp.int32),
            pltpu.VMEM((cap,), jnp.int32),
            pltpu.VMEM((B, D), jnp.float32),
            pltpu.VMEM((B, D), jnp.float32),
            pltpu.SMEM((8,), jnp.int32),
            pltpu.VMEM_SHARED((W + L, D), jnp.float32),
        ],
    )


def _gsum_sc(Eu, n):
    chunk = Eu // (NT * NC)
    nfull = chunk // B
    rem = chunk - nfull * B
    assert rem % 8 == 0

    def body(xa_hbm, xb_hbm, i_hbm, j_hbm, out_hbm, i_t, j_t, arows, brows):
        c = lax.axis_index("c")
        s = lax.axis_index("s")
        wid = s * NC + c
        base = wid * chunk
        pltpu.sync_copy(i_hbm.at[pl.ds(base, chunk)], i_t)
        pltpu.sync_copy(j_hbm.at[pl.ds(base, chunk)], j_t)

        def do_batch(goff, bsz):
            pltpu.sync_copy(xa_hbm.at[i_t.at[pl.ds(goff, bsz)]], arows.at[pl.ds(0, bsz)])
            pltpu.sync_copy(xb_hbm.at[j_t.at[pl.ds(goff, bsz)]], brows.at[pl.ds(0, bsz)])

            def rowfn(r, _):
                for k in range(D // L):
                    sl = pl.ds(k * L, L)
                    arows[r, sl] = arows[r, sl] + brows[r, sl]
                return 0
            lax.fori_loop(0, bsz, rowfn, 0)
            pltpu.sync_copy(arows.at[pl.ds(0, bsz)], out_hbm.at[pl.ds(base + goff, bsz)])

        def batch(g, _):
            do_batch(g * B, B)
            return 0
        lax.fori_loop(0, nfull, batch, 0)
        if rem:
            do_batch(nfull * B, rem)

    return pl.kernel(
        body,
        out_type=jax.ShapeDtypeStruct((Eu, D), jnp.float32),
        mesh=_mesh(),
        compiler_params=_params(),
        scratch_types=[
            pltpu.VMEM((chunk,), jnp.int32),
            pltpu.VMEM((chunk,), jnp.int32),
            pltpu.VMEM((B, D), jnp.float32),
            pltpu.VMEM((B, D), jnp.float32),
        ],
    )


def _rows_spec(R):
    return pl.BlockSpec((R, D), lambda g: (g, 0))


def _full_spec(shape):
    return pl.BlockSpec(shape, lambda g: tuple(0 for _ in shape))


_DN = (((1,), (1,)), ((), ()))


def _matmul3_tc(n, R):
    def body(x_ref, wd, wa, wb_, bd, ba, bb, od, oa, ob):
        x = x_ref[...]
        od[...] = lax.dot_general(x, wd[...], _DN, preferred_element_type=jnp.float32) + bd[...]
        oa[...] = lax.dot_general(x, wa[...], _DN, preferred_element_type=jnp.float32) + ba[...]
        ob[...] = lax.dot_general(x, wb_[...], _DN, preferred_element_type=jnp.float32) + bb[...]

    return pl.pallas_call(
        body,
        grid=(n // R,),
        in_specs=[_rows_spec(R)] + [_full_spec((D, D))] * 3 + [_full_spec((1, D))] * 3,
        out_specs=[_rows_spec(R)] * 3,
        out_shape=[jax.ShapeDtypeStruct((n, D), jnp.float32)] * 3,
    )


def _sigmoid_tc(n, R):
    def body(x_ref, o_ref):
        o_ref[...] = jax.nn.sigmoid(x_ref[...])
    return pl.pallas_call(
        body,
        grid=(n // R,),
        in_specs=[_rows_spec(R)],
        out_specs=_rows_spec(R),
        out_shape=jax.ShapeDtypeStruct((n, D), jnp.float32),
    )


def _stats_tc(n, R):
    def body(a_ref, badd_ref, w_ref, b_ref, o_ref):
        pre = lax.dot_general(a_ref[...], w_ref[...], _DN,
                              preferred_element_type=jnp.float32) + b_ref[...] + badd_ref[...]
        s1 = jnp.sum(pre, axis=0, keepdims=True)
        s2 = jnp.sum(pre * pre, axis=0, keepdims=True)
        blk = jnp.concatenate([s1, s2, jnp.zeros((6, D), jnp.float32)], axis=0)

        @pl.when(pl.program_id(0) == 0)
        def _():
            o_ref[...] = jnp.zeros_like(o_ref)
        o_ref[...] += blk

    return pl.pallas_call(
        body,
        grid=(n // R,),
        in_specs=[_rows_spec(R), _rows_spec(R), _full_spec((D, D)), _full_spec((1, D))],
        out_specs=_full_spec((8, D)),
        out_shape=jax.ShapeDtypeStruct((8, D), jnp.float32),
    )


def _finalize_tc(n, R, with_sigma):
    def body(a_ref, badd_ref, w_ref, b_ref, st_ref, g_ref, b2_ref, *outs):
        a = a_ref[...]
        pre = lax.dot_general(a, w_ref[...], _DN,
                              preferred_element_type=jnp.float32) + b_ref[...] + badd_ref[...]
        mu = st_ref[0:1, :] * (1.0 / n)
        var = st_ref[1:2, :] * (1.0 / n) - mu * mu
        inv = lax.rsqrt(var + 1e-5)
        y = g_ref[...] * (pre - mu) * inv + b2_ref[...]
        out = a + y * jax.nn.sigmoid(y)
        outs[0][...] = out
        if with_sigma:
            outs[1][...] = jax.nn.sigmoid(out)

    n_out = 2 if with_sigma else 1
    return pl.pallas_call(
        body,
        grid=(n // R,),
        in_specs=[_rows_spec(R), _rows_spec(R), _full_spec((D, D)), _full_spec((1, D)),
                  _full_spec((8, D)), _full_spec((1, D)), _full_spec((1, D))],
        out_specs=[_rows_spec(R)] * n_out,
        out_shape=[jax.ShapeDtypeStruct((n, D), jnp.float32)] * n_out,
    )


def _egg_pallas(x, ea, i, j, p, W, P, emit_sigma, sigma):
    (Wsw, Wsb, Wdw, Wdb, Waw, Wab, Wbw, Wbb, Wcw, Wcb, gn, bn_, ge, be) = p
    n = x.shape[0]
    Eu = ea.shape[0]
    Rn = 2000
    Re = 2000
    r1 = lambda v: v.reshape(1, D)

    if sigma is None:
        sigma = _sigmoid_tc(Eu, Re)(ea)
    xd, xa, xb = _matmul3_tc(n, Rn)(x, Wdw, Waw, Wbw, r1(Wdb), r1(Wab), r1(Wbb))
    ssum = _seg_sum_sc(Eu, n, W, P)(sigma, i)
    dest = _dest_sc(Eu, n, W, P)(sigma, i, j, ssum, xd)
    st_n = _stats_tc(n, Rn)(x, dest, Wsw, r1(Wsb))
    fin_n = _finalize_tc(n, Rn, emit_sigma)(x, dest, Wsw, r1(Wsb), st_n, r1(gn), r1(bn_))
    if emit_sigma:
        new_x, sig_next = fin_n
    else:
        new_x, sig_next = fin_n[0], None
    gsum = _gsum_sc(Eu, n)(xa, xb, i, j)
    st_e = _stats_tc(Eu, Re)(ea, gsum, Wcw, r1(Wcb))
    new_ea = _finalize_tc(Eu, Re, False)(ea, gsum, Wcw, r1(Wcb), st_e, r1(ge), r1(be))[0]
    return new_x, new_ea, sig_next


PN = ["Wsrc_w", "Wsrc_b", "Wdst_w", "Wdst_b", "Wai_w", "Wai_b", "Wbj_w",
      "Wbj_b", "Wcij_w", "Wcij_b", "bn_n_g", "bn_n_b", "bn_e_g", "bn_e_b"]


def kernel(node_feats, edge_attr, triplet_feats, edge_index, edge_index_lg, eu_Wsrc_w, eu_Wsrc_b, eu_Wdst_w, eu_Wdst_b, eu_Wai_w, eu_Wai_b, eu_Wbj_w, eu_Wbj_b, eu_Wcij_w, eu_Wcij_b, eu_bn_n_g, eu_bn_n_b, eu_bn_e_g, eu_bn_e_b, nu_Wsrc_w, nu_Wsrc_b, nu_Wdst_w, nu_Wdst_b, nu_Wai_w, nu_Wai_b, nu_Wbj_w, nu_Wbj_b, nu_Wcij_w, nu_Wcij_b, nu_bn_n_g, nu_bn_n_b, nu_bn_e_g, nu_bn_e_b):
    kw = dict(locals())
    pe = tuple(kw["eu_" + nm] for nm in PN)
    pn = tuple(kw["nu_" + nm] for nm in PN)
    i1 = edge_index_lg[0].astype(jnp.int32)
    j1 = edge_index_lg[1].astype(jnp.int32)
    i2 = edge_index[0].astype(jnp.int32)
    j2 = edge_index[1].astype(jnp.int32)

    message, trip, sig2 = _egg_pallas(edge_attr, triplet_feats, i1, j1, pe,
                                      W=10000, P=8, emit_sigma=True, sigma=None)
    nf, ea_out, _ = _egg_pallas(node_feats, message, i2, j2, pn,
                                W=5000, P=1, emit_sigma=False, sigma=sig2)
    return (nf, ea_out, trip)
